# baseline (speedup 1.0000x reference)
import jax
import jax.numpy as jnp
from jax import lax
from jax.experimental import pallas as pl
from jax.experimental.pallas import tpu as pltpu
from jax.experimental.pallas import tpu_sc as plsc

_NC = 2
_NS = 16
_NW = _NC * _NS
_L = 16
_BR = 8
_BC = 512
_NV = (_BR * _BC) // _L
_D = 10


def _sc_body(x_hbm, bins_hbm, out_hbm,
             xb0, xb1, pb0, pb1, ib0, ib1, onesb, binsb, zb,
             si0, si1, so0, so1):
    M, N = x_hbm.shape
    rows_per_w = M // _NW
    bands = rows_per_w // _BR
    ccs = N // _BC
    pairs = (bands * ccs) // 2
    wid = lax.axis_index("s") * _NC + lax.axis_index("c")
    rbase = wid * rows_per_w

    pltpu.sync_copy(bins_hbm, binsb.at[pl.ds(0, 10)])
    vb = binsb[pl.ds(0, _L)]
    thr = [jnp.broadcast_to(vb[k], (_L,)) for k in range(1, 9)]
    vone = jnp.full((_L,), 1.0, jnp.float32)
    vzero = jnp.zeros((_L,), jnp.float32)
    viota = lax.iota(jnp.int32, _L)

    def _init2(v, _):
        onesb[v // (_BC // _L), pl.ds((v % (_BC // _L)) * _L, _L)] = vone
        return 0
    lax.fori_loop(0, _NV, _init2, 0)
    vo = onesb[0, pl.ds(0, _L)]
    zb[pl.ds(0, _L)] = vzero
    vz = zb[pl.ds(0, _L)]

    def _zero3(v, _):
        c = v // (_BR * (_BC // _L))
        rem = v % (_BR * (_BC // _L))
        rw = rem // (_BC // _L)
        cl = (rem % (_BC // _L)) * _L
        pb0[c, rw, pl.ds(cl, _L)] = vzero
        pb1[c, rw, pl.ds(cl, _L)] = vzero
        return 0
    lax.fori_loop(0, 9 * _BR * (_BC // _L), _zero3, 0)

    def _in_copy(xb, si, r0, c0):
        return pltpu.make_async_copy(
            x_hbm.at[pl.ds(r0, _BR), pl.ds(c0, _BC)], xb, si)

    def _out_copies(pbk, so, r0, c0):
        cps = [pltpu.make_async_copy(
                   pbk.at[c],
                   out_hbm.at[c + 1, pl.ds(r0, _BR), pl.ds(c0, _BC)],
                   so)
               for c in range(9)]
        cps.append(pltpu.make_async_copy(
            onesb, out_hbm.at[0, pl.ds(r0, _BR), pl.ds(c0, _BC)], so))
        return cps

    _in_copy(xb0, si0, rbase, 0).start()

    def _chunk(t2, parity, xb, pbk, ibk, si, so):
        t = 2 * t2 + parity
        band = t // ccs
        cj = t % ccs
        r0 = rbase + band * _BR
        c0 = cj * _BC

        tn = t + 1
        if parity == 0:
            bn = tn // ccs
            _in_copy(xb1, si1, rbase + bn * _BR, (tn % ccs) * _BC).start()
        else:
            @pl.when(tn < 2 * pairs)
            def _():
                bn = tn // ccs
                _in_copy(xb0, si0, rbase + bn * _BR, (tn % ccs) * _BC).start()

        @pl.when(t2 > 0)
        def _():
            for cp in _out_copies(pbk, so, r0, c0):
                cp.wait()

            def _clear(v, _):
                accv = ibk[pl.ds(v * _L, _L)]
                rw = v // (_BC // _L)
                colv = (v % (_BC // _L)) * _L + viota
                rwv = jnp.broadcast_to(rw, (_L,))
                plsc.store_scatter(pbk, [accv, rwv, colv], vz)
                return 0
            lax.fori_loop(0, _NV, _clear, 0)

        _in_copy(xb, si, r0, c0).wait()

        def _compute(v, _):
            rw = v // (_BC // _L)
            cl = (v % (_BC // _L)) * _L
            xv = xb[rw, pl.ds(cl, _L)]
            acc = jnp.where(xv > thr[0], 1, 0)
            for th in thr[1:]:
                acc = acc + jnp.where(xv > th, 1, 0)
            ibk[pl.ds(v * _L, _L)] = acc
            rwv = jnp.broadcast_to(rw, (_L,))
            colv = cl + viota
            plsc.store_scatter(pbk, [acc, rwv, colv], vo)
            return 0
        lax.fori_loop(0, _NV, _compute, 0)

        for cp in _out_copies(pbk, so, r0, c0):
            cp.start()

    def _pair(t2, _):
        _chunk(t2, 0, xb0, pb0, ib0, si0, so0)
        _chunk(t2, 1, xb1, pb1, ib1, si1, so1)
        return 0
    lax.fori_loop(0, pairs, _pair, 0)

    lastb = (2 * pairs - 2) // ccs
    for cp in _out_copies(pb0, so0, rbase + lastb * _BR, ((2 * pairs - 2) % ccs) * _BC):
        cp.wait()
    lastb = (2 * pairs - 1) // ccs
    for cp in _out_copies(pb1, so1, rbase + lastb * _BR, ((2 * pairs - 1) % ccs) * _BC):
        cp.wait()


def kernel(x, bins):
    M, N = x.shape
    mesh = plsc.VectorSubcoreMesh(core_axis_name="c", subcore_axis_name="s")
    run = pl.kernel(
        _sc_body,
        out_type=jax.ShapeDtypeStruct((_D, M, N), jnp.float32),
        mesh=mesh,
        scratch_types=[
            pltpu.VMEM((_BR, _BC), jnp.float32),
            pltpu.VMEM((_BR, _BC), jnp.float32),
            pltpu.VMEM((9, _BR, _BC), jnp.float32),
            pltpu.VMEM((9, _BR, _BC), jnp.float32),
            pltpu.VMEM((_BR * _BC,), jnp.int32),
            pltpu.VMEM((_BR * _BC,), jnp.int32),
            pltpu.VMEM((_BR, _BC), jnp.float32),
            pltpu.VMEM((16,), jnp.float32),
            pltpu.VMEM((16,), jnp.float32),
            pltpu.SemaphoreType.DMA,
            pltpu.SemaphoreType.DMA,
            pltpu.SemaphoreType.DMA,
            pltpu.SemaphoreType.DMA,
        ],
        compiler_params=pltpu.CompilerParams(needs_layout_passes=False),
    )
    out = run(x, bins)
    return jnp.transpose(out, (1, 2, 0))


# speedup vs baseline: 1.0865x; 1.0865x over previous
import functools

import jax
import jax.numpy as jnp
from jax import lax
from jax.experimental import pallas as pl
from jax.experimental.pallas import tpu as pltpu
from jax.experimental.pallas import tpu_sc as plsc

_NC = 2
_NS = 16
_NW = _NC * _NS
_L = 16
_CH = 4096
_CHV = _CH // _L
_D = 10


def _sc_body(x_hbm, bins_hbm, out_hbm,
             xb0, xb1, pb0, pb1, ib0, ib1, onesb, binsb, zb,
             si0, si1, so0, so1):
    M = x_hbm.shape[0]
    rows_per_w = M // _NW
    wid = lax.axis_index("s") * _NC + lax.axis_index("c")
    rbase = wid * rows_per_w

    pltpu.sync_copy(bins_hbm, binsb.at[pl.ds(0, 10)])
    vb = binsb[pl.ds(0, _L)]
    thr = [jnp.broadcast_to(vb[k], (_L,)) for k in range(1, 9)]
    vone = jnp.full((_L,), 1.0, jnp.float32)
    vzero = jnp.zeros((_L,), jnp.float32)
    viota = lax.iota(jnp.int32, _L)

    def _init(v, _):
        onesb[pl.ds(v * _L, _L)] = vone
        return 0
    lax.fori_loop(0, _CHV, _init, 0)
    vo = onesb[pl.ds(0, _L)]
    zb[pl.ds(0, _L)] = vzero
    vz = zb[pl.ds(0, _L)]

    def _zero(v, _):
        pb0[pl.ds(v * _L, _L)] = vzero
        pb1[pl.ds(v * _L, _L)] = vzero
        return 0
    lax.fori_loop(0, 9 * _CHV, _zero, 0)

    def _in_copy(xb, si, r, off):
        return pltpu.make_async_copy(x_hbm.at[r, pl.ds(off, _CH)], xb, si)

    def _out_copies(pbk, so, r, off):
        cps = [pltpu.make_async_copy(pbk.at[pl.ds(c * _CH, _CH)],
                                     out_hbm.at[c + 1, r, pl.ds(off, _CH)],
                                     so)
               for c in range(9)]
        cps.append(pltpu.make_async_copy(onesb, out_hbm.at[0, r, pl.ds(off, _CH)], so))
        return cps

    _in_copy(xb0, si0, rbase, 0).start()

    def _chunk(r, k, xb, pbk, ibk, si, so):
        off = k * _CH

        if k == 0:
            _in_copy(xb1, si1, r, _CH).start()
        else:
            @pl.when(r + 1 < rbase + rows_per_w)
            def _():
                _in_copy(xb0, si0, r + 1, 0).start()

        @pl.when(r > rbase)
        def _():
            for cp in _out_copies(pbk, so, r, off):
                cp.wait()

            def _clear(v, _):
                sidx = ibk[pl.ds(v * _L, _L)]
                plsc.store_scatter(pbk, [sidx], vz)
                return 0
            lax.fori_loop(0, _CHV, _clear, 0)

        _in_copy(xb, si, r, off).wait()

        def _compute(v, _):
            xv = xb[pl.ds(v * _L, _L)]
            acc = jnp.where(xv > thr[0], 1, 0)
            for t in thr[1:]:
                acc = acc + jnp.where(xv > t, 1, 0)
            sidx = acc * _CH + (v * _L + viota)
            ibk[pl.ds(v * _L, _L)] = sidx
            plsc.store_scatter(pbk, [sidx], vo)
            return 0
        lax.fori_loop(0, _CHV, _compute, 0)

        for cp in _out_copies(pbk, so, r, off):
            cp.start()

    def _row(r, _):
        _chunk(r, 0, xb0, pb0, ib0, si0, so0)
        _chunk(r, 1, xb1, pb1, ib1, si1, so1)
        return 0
    lax.fori_loop(rbase, rbase + rows_per_w, _row, 0)

    last = rbase + rows_per_w - 1
    for cp in _out_copies(pb0, so0, last, 0):
        cp.wait()
    for cp in _out_copies(pb1, so1, last, _CH):
        cp.wait()


def kernel(x, bins):
    M, N = x.shape
    mesh = plsc.VectorSubcoreMesh(core_axis_name="c", subcore_axis_name="s")
    run = pl.kernel(
        _sc_body,
        out_type=jax.ShapeDtypeStruct((_D, M, N), jnp.float32),
        mesh=mesh,
        scratch_types=[
            pltpu.VMEM((_CH,), jnp.float32),
            pltpu.VMEM((_CH,), jnp.float32),
            pltpu.VMEM((9 * _CH,), jnp.float32),
            pltpu.VMEM((9 * _CH,), jnp.float32),
            pltpu.VMEM((_CH,), jnp.int32),
            pltpu.VMEM((_CH,), jnp.int32),
            pltpu.VMEM((_CH,), jnp.float32),
            pltpu.VMEM((16,), jnp.float32),
            pltpu.VMEM((16,), jnp.float32),
            pltpu.SemaphoreType.DMA,
            pltpu.SemaphoreType.DMA,
            pltpu.SemaphoreType.DMA,
            pltpu.SemaphoreType.DMA,
        ],
        compiler_params=pltpu.CompilerParams(needs_layout_passes=False),
    )
    out = run(x, bins)
    return jnp.transpose(out, (1, 2, 0))


# speedup vs baseline: 2.3945x; 2.2038x over previous
import jax
import jax.numpy as jnp
from jax import lax
from jax.experimental import pallas as pl
from jax.experimental.pallas import tpu as pltpu
from jax.experimental.pallas import tpu_sc as plsc

_NC = 2
_NS = 16
_NW = _NC * _NS
_L = 16
_CH = 4096
_CHV = _CH // _L
_D = 10


def _sc_body(x_hbm, bins_hbm, out_hbm,
             xb0, xb1, pb0, pb1, ib0, ib1, onesb, binsb, zb,
             si0, si1, so0, so1):
    M = x_hbm.shape[0]
    rows_per_w = M // _NW
    wid = lax.axis_index("s") * _NC + lax.axis_index("c")
    rbase = wid * rows_per_w

    pltpu.sync_copy(bins_hbm, binsb.at[pl.ds(0, 10)])
    vb = binsb[pl.ds(0, _L)]
    thr = [jnp.broadcast_to(vb[k], (_L,)) for k in range(1, 9)]
    vone = jnp.full((_L,), 1.0, jnp.float32)
    vzero = jnp.zeros((_L,), jnp.float32)
    viota = lax.iota(jnp.int32, _L)

    @plsc.parallel_loop(0, _CHV, 1, unroll=8)
    def _(v):
        onesb[pl.ds(v * _L, _L)] = vone

    vo = onesb[pl.ds(0, _L)]
    zb[pl.ds(0, _L)] = vzero
    vz = zb[pl.ds(0, _L)]

    @plsc.parallel_loop(0, 9 * _CHV, 1, unroll=8)
    def _(v):
        pb0[pl.ds(v * _L, _L)] = vzero
        pb1[pl.ds(v * _L, _L)] = vzero

    def _in_copy(xb, si, r, off):
        return pltpu.make_async_copy(x_hbm.at[r, pl.ds(off, _CH)], xb, si)

    def _out_copies(pbk, so, r, off):
        cps = [pltpu.make_async_copy(pbk.at[pl.ds(c * _CH, _CH)],
                                     out_hbm.at[c + 1, r, pl.ds(off, _CH)],
                                     so)
               for c in range(9)]
        cps.append(pltpu.make_async_copy(onesb, out_hbm.at[0, r, pl.ds(off, _CH)], so))
        return cps

    _in_copy(xb0, si0, rbase, 0).start()

    def _bucket(xv):
        acc = jnp.where(xv > thr[0], 1, 0)
        for th in thr[1:]:
            acc = acc + jnp.where(xv > th, 1, 0)
        return acc

    def _chunk(r, k, xb, pbk, ibk, si, so):
        off = k * _CH

        if k == 0:
            _in_copy(xb1, si1, r, _CH).start()
        else:
            @pl.when(r + 1 < rbase + rows_per_w)
            def _():
                _in_copy(xb0, si0, r + 1, 0).start()

        @pl.when(r > rbase)
        def _():
            for cp in _out_copies(pbk, so, r, off):
                cp.wait()

        _in_copy(xb, si, r, off).wait()

        @pl.when(r > rbase)
        def _():
            @plsc.parallel_loop(0, _CHV, 1, unroll=8)
            def _(v):
                old = ibk[pl.ds(v * _L, _L)]
                xv = xb[pl.ds(v * _L, _L)]
                new = _bucket(xv) * _CH + (v * _L + viota)
                ibk[pl.ds(v * _L, _L)] = new
                plsc.store_scatter(pbk, [new], vo)
                plsc.store_scatter(pbk, [old], vz, mask=old != new)

        @pl.when(r == rbase)
        def _():
            @plsc.parallel_loop(0, _CHV, 1, unroll=8)
            def _(v):
                xv = xb[pl.ds(v * _L, _L)]
                new = _bucket(xv) * _CH + (v * _L + viota)
                ibk[pl.ds(v * _L, _L)] = new
                plsc.store_scatter(pbk, [new], vo)

        for cp in _out_copies(pbk, so, r, off):
            cp.start()

    def _row(r, _):
        _chunk(r, 0, xb0, pb0, ib0, si0, so0)
        _chunk(r, 1, xb1, pb1, ib1, si1, so1)
        return 0
    lax.fori_loop(rbase, rbase + rows_per_w, _row, 0)

    last = rbase + rows_per_w - 1
    for cp in _out_copies(pb0, so0, last, 0):
        cp.wait()
    for cp in _out_copies(pb1, so1, last, _CH):
        cp.wait()


def kernel(x, bins):
    M, N = x.shape
    mesh = plsc.VectorSubcoreMesh(core_axis_name="c", subcore_axis_name="s")
    run = pl.kernel(
        _sc_body,
        out_type=jax.ShapeDtypeStruct((_D, M, N), jnp.float32),
        mesh=mesh,
        scratch_types=[
            pltpu.VMEM((_CH,), jnp.float32),
            pltpu.VMEM((_CH,), jnp.float32),
            pltpu.VMEM((9 * _CH,), jnp.float32),
            pltpu.VMEM((9 * _CH,), jnp.float32),
            pltpu.VMEM((_CH,), jnp.int32),
            pltpu.VMEM((_CH,), jnp.int32),
            pltpu.VMEM((_CH,), jnp.float32),
            pltpu.VMEM((16,), jnp.float32),
            pltpu.VMEM((16,), jnp.float32),
            pltpu.SemaphoreType.DMA,
            pltpu.SemaphoreType.DMA,
            pltpu.SemaphoreType.DMA,
            pltpu.SemaphoreType.DMA,
        ],
        compiler_params=pltpu.CompilerParams(needs_layout_passes=False),
    )
    out = run(x, bins)
    return jnp.transpose(out, (1, 2, 0))
